# baseline (speedup 1.0000x reference)
import functools

import jax
import jax.numpy as jnp
from jax import lax
from jax.experimental import pallas as pl
from jax.experimental.pallas import tpu as pltpu
from jax.experimental.pallas import tpu_sc as plsc

_D0, _N, _D2, _C = 2, 1024, 16, 1024
_FLAT = _D0 * _N * _D2 * _C
_NCHUNK = 16
_CH = _FLAT // _NCHUNK
_NC, _NS = 2, 16
_NW = _NC * _NS
_KPW = _N // _NW


@functools.partial(
    pl.kernel,
    out_type=(),
    mesh=pltpu.create_tensorcore_mesh("x", num_cores=1),
    scratch_types=[pltpu.SemaphoreType.DMA],
)
def _tc_copy_into(t_hbm, out_ref, sem):
    copies = [
        pltpu.async_copy(t_hbm.at[pl.ds(i * _CH, _CH)],
                         out_ref.at[pl.ds(i * _CH, _CH)], sem)
        for i in range(_NCHUNK)
    ]
    for c in copies:
        c.wait()


@functools.partial(
    pl.kernel,
    out_type=(),
    mesh=plsc.VectorSubcoreMesh(core_axis_name="c", subcore_axis_name="s"),
    scratch_types=[
        pltpu.VMEM((_N,), jnp.int32),
        pltpu.VMEM((_D0 * _D2, _N), jnp.float32),
        pltpu.SemaphoreType.DMA,
    ],
)
def _sc_diag_scatter(out_ref, idx_hbm, vt_hbm, idx_v, v_v, sem):
    wid = lax.axis_index("s") * _NC + lax.axis_index("c")
    base_k = wid * _KPW
    pltpu.sync_copy(idx_hbm, idx_v)
    pltpu.sync_copy(vt_hbm, v_v)
    copies = []
    for g in range(_KPW // 16):
        idx16 = idx_v[pl.ds(base_k + g * 16, 16)]
        base16 = idx16 * (_D2 * _C + 1)
        for d0 in range(_D0):
            for d2 in range(_D2):
                off16 = base16 + (d0 * _N * _D2 * _C + d2 * _C)
                copies.append(pltpu.async_copy(
                    v_v.at[d0 * _D2 + d2, pl.ds(base_k + g * 16, 16)],
                    out_ref.at[off16], sem))
    for c in copies:
        c.wait()


@functools.partial(jax.jit, static_argnames=())
def kernel(t, idx, v):
    out_ref = jax.new_ref(pl.empty((_FLAT,), jnp.float32),
                          memory_space=pltpu.HBM)
    _tc_copy_into(t.reshape(_FLAT), out_ref)
    idx32 = idx.astype(jnp.int32)
    vt = v.transpose(1, 2, 0).reshape(_D0 * _D2, _N)
    _sc_diag_scatter(out_ref, idx32, vt)
    return out_ref[...].reshape(_D0, _N, _D2, _C)


# speedup vs baseline: 50.2477x; 50.2477x over previous
import functools

import jax
import jax.numpy as jnp
from jax.experimental import pallas as pl

_D0, _N, _D2, _C = 2, 1024, 16, 1024
_R = 128


def _diag_set_kernel(idx_ref, v_ref, t_ref, o_ref):
    i = pl.program_id(1)
    tb = t_ref[...]
    rows = i * _R + jax.lax.broadcasted_iota(jnp.int32, (_R, 1), 0)
    eq = idx_ref[...] == rows
    member = eq.any(axis=1)
    vsel = jnp.dot(eq.astype(jnp.float32), v_ref[0],
                   preferred_element_type=jnp.float32,
                   precision=jax.lax.Precision.HIGHEST)
    col = jax.lax.broadcasted_iota(jnp.int32, (1, _R, _D2, _C), 3)
    rowg = jax.lax.broadcasted_iota(jnp.int32, (1, _R, _D2, _C), 1) + i * _R
    mask = (col == rowg) & member[None, :, None, None]
    o_ref[...] = jnp.where(mask, vsel[None, :, :, None], tb)


@functools.partial(jax.jit, static_argnames=())
def kernel(t, idx, v):
    idx2 = idx.reshape(1, _N).astype(jnp.int32)
    v2 = v.transpose(1, 0, 2)
    grid = (_D0, _N // _R)
    return pl.pallas_call(
        _diag_set_kernel,
        grid=grid,
        in_specs=[
            pl.BlockSpec((1, _N), lambda j, i: (0, 0)),
            pl.BlockSpec((1, _N, _D2), lambda j, i: (j, 0, 0)),
            pl.BlockSpec((1, _R, _D2, _C), lambda j, i: (j, i, 0, 0)),
        ],
        out_specs=pl.BlockSpec((1, _R, _D2, _C), lambda j, i: (j, i, 0, 0)),
        out_shape=jax.ShapeDtypeStruct(t.shape, t.dtype),
    )(idx2, v2, t)


# speedup vs baseline: 50.6366x; 1.0077x over previous
import functools

import jax
import jax.numpy as jnp
from jax.experimental import pallas as pl

_D0, _N, _D2, _C = 2, 1024, 16, 1024
_R = 64


def _diag_set_kernel(idx_ref, v_ref, t_ref, o_ref):
    i = pl.program_id(0)
    tb = t_ref[...]
    rows = i * _R + jax.lax.broadcasted_iota(jnp.int32, (_R, 1), 0)
    eq = idx_ref[...] == rows
    member = eq.any(axis=1)
    vsel = jnp.dot(eq.astype(jnp.float32), v_ref[...],
                   preferred_element_type=jnp.float32,
                   precision=jax.lax.Precision.HIGHEST)
    vsel = vsel.reshape(_R, _D0, _D2).transpose(1, 0, 2)
    col = jax.lax.broadcasted_iota(jnp.int32, (_D0, _R, _D2, _C), 3)
    rowg = jax.lax.broadcasted_iota(jnp.int32, (_D0, _R, _D2, _C), 1) + i * _R
    mask = (col == rowg) & member[None, :, None, None]
    o_ref[...] = jnp.where(mask, vsel[..., None], tb)


@functools.partial(jax.jit, static_argnames=())
def kernel(t, idx, v):
    idx2 = idx.reshape(1, _N).astype(jnp.int32)
    v2 = v.reshape(_N, _D0 * _D2)
    grid = (_N // _R,)
    return pl.pallas_call(
        _diag_set_kernel,
        grid=grid,
        in_specs=[
            pl.BlockSpec((1, _N), lambda i: (0, 0)),
            pl.BlockSpec((_N, _D0 * _D2), lambda i: (0, 0)),
            pl.BlockSpec((_D0, _R, _D2, _C), lambda i: (0, i, 0, 0)),
        ],
        out_specs=pl.BlockSpec((_D0, _R, _D2, _C), lambda i: (0, i, 0, 0)),
        out_shape=jax.ShapeDtypeStruct(t.shape, t.dtype),
    )(idx2, v2, t)
